# Initial kernel scaffold; baseline (speedup 1.0000x reference)
import jax
import jax.numpy as jnp
from jax.experimental import pallas as pl


def kernel(z, raw_weights, alpha, beta, bias):
    raise NotImplementedError("write your pallas kernel here")


# speedup vs baseline: 98.3820x; 98.3820x over previous
import functools

import jax
import jax.numpy as jnp
import numpy as np
from jax import lax
from jax.experimental import pallas as pl
from jax.experimental.pallas import tpu as pltpu
from jax.experimental.pallas import tpu_sc as plsc

NUM_BASIS = 16
GRID_SIZE = 1000
Z_MIN = -8.0
Z_MAX = 8.0
TABLE_PAD = 1024

U_SCALE = np.float32((GRID_SIZE - 1) / (Z_MAX - Z_MIN + 1e-8))
U_MAX = np.float32(GRID_SIZE - 1 - 1e-6)


def _make_I_grid_T():
    z_grid = jnp.linspace(Z_MIN, Z_MAX, GRID_SIZE)
    knots = jnp.linspace(Z_MIN, Z_MAX, NUM_BASIS)
    d = jnp.abs(z_grid[:, None] - knots[None, :])
    dx = (Z_MAX - Z_MIN) / (NUM_BASIS - 1)
    H = jnp.clip(1.0 - d / dx, 0.0, None)
    H = H / (H.sum(axis=1, keepdims=True) + 1e-08)
    dz = z_grid[1] - z_grid[0]
    I = jnp.cumsum(H * dz, axis=0)
    I_max = I[-1, :]
    I_max = jnp.where(I_max <= 0, 1.0, I_max)
    I = I / I_max[None, :]
    I_T = I.astype(jnp.float32).T
    return jnp.pad(I_T, ((0, 0), (0, TABLE_PAD - GRID_SIZE)))


def _table_body(ig_ref, rw_ref, pb_ref, out_ref):
    w = jax.nn.softplus(rw_ref[...])
    s = jnp.sum(ig_ref[...] * w, axis=0, keepdims=True)
    out_ref[...] = s * pb_ref[0] + pb_ref[1]


def _build_table(ig_T, raw_weights, beta, bias):
    pb = jnp.concatenate([beta.reshape(1), bias.reshape(1)]).astype(jnp.float32)
    out = pl.pallas_call(
        _table_body,
        out_shape=jax.ShapeDtypeStruct((1, TABLE_PAD), jnp.float32),
        in_specs=[
            pl.BlockSpec(memory_space=pltpu.VMEM),
            pl.BlockSpec(memory_space=pltpu.VMEM),
            pl.BlockSpec(memory_space=pltpu.SMEM),
        ],
        out_specs=pl.BlockSpec(memory_space=pltpu.VMEM),
    )(ig_T, raw_weights.reshape(NUM_BASIS, 1), pb)
    return out.reshape(TABLE_PAD)


def _make_sc_kernel(n_total):
    info = plsc.get_sparse_core_info()
    nc, ns, nl = info.num_cores, info.num_subcores, info.num_lanes
    nw = nc * ns
    per_w = n_total // nw
    chunk = 16384
    nchunk = per_w // chunk
    mesh = plsc.VectorSubcoreMesh(core_axis_name="c", subcore_axis_name="s")

    @functools.partial(
        pl.kernel,
        mesh=mesh,
        out_type=jax.ShapeDtypeStruct((n_total,), jnp.float32),
        scratch_types=[
            pltpu.VMEM((TABLE_PAD,), jnp.float32),
            pltpu.VMEM((nl,), jnp.float32),
            pltpu.VMEM((chunk,), jnp.float32),
            pltpu.VMEM((chunk,), jnp.float32),
        ],
        compiler_params=pltpu.CompilerParams(needs_layout_passes=False),
    )
    def sck(tab_hbm, alpha_hbm, z_hbm, out_hbm, tab_v, alpha_v, zb, ob):
        wid = lax.axis_index("s") * nc + lax.axis_index("c")
        base = wid * per_w
        pltpu.sync_copy(tab_hbm, tab_v)
        pltpu.sync_copy(alpha_hbm, alpha_v)
        av = alpha_v[...]

        def chunk_body(c, carry):
            off = base + c * chunk
            pltpu.sync_copy(z_hbm.at[pl.ds(off, chunk)], zb)

            def vec_body(i, carry2):
                zv = zb[pl.ds(i * nl, nl)]
                zc = jnp.minimum(jnp.maximum(zv, Z_MIN), Z_MAX)
                u = (zc - Z_MIN) * U_SCALE
                u = jnp.minimum(jnp.maximum(u, 0.0), U_MAX)
                idx = u.astype(jnp.int32)
                fr = u - idx.astype(jnp.float32)
                t0 = plsc.load_gather(tab_v, [idx])
                t1 = plsc.load_gather(tab_v, [idx + 1])
                g = av * zc + t0 + fr * (t1 - t0)
                ob[pl.ds(i * nl, nl)] = 1.0 / (1.0 + jnp.exp(-g))
                return carry2

            lax.fori_loop(0, chunk // nl, vec_body, 0, unroll=4)
            pltpu.sync_copy(ob, out_hbm.at[pl.ds(off, chunk)])
            return carry

        lax.fori_loop(0, nchunk, chunk_body, 0)

    return sck


def kernel(z, raw_weights, alpha, beta, bias):
    orig_shape = z.shape
    n_total = int(np.prod(orig_shape))
    table = _build_table(_make_I_grid_T(), raw_weights, beta, bias)
    alpha_vec = jnp.full((16,), alpha, dtype=jnp.float32)
    sck = _make_sc_kernel(n_total)
    out = sck(table, alpha_vec, z.reshape(n_total))
    return out.reshape(orig_shape)


# speedup vs baseline: 361.5415x; 3.6749x over previous
import functools

import jax
import jax.numpy as jnp
import numpy as np
from jax import lax
from jax.experimental import pallas as pl
from jax.experimental.pallas import tpu as pltpu
from jax.experimental.pallas import tpu_sc as plsc

NUM_BASIS = 16
GRID_SIZE = 1000
Z_MIN = -8.0
Z_MAX = 8.0
TABLE_PAD = 1024

U_SCALE = np.float32((GRID_SIZE - 1) / (Z_MAX - Z_MIN + 1e-8))
U_MAX = np.float32(GRID_SIZE - 1 - 1e-6)


def _make_I_grid_T():
    z_grid = jnp.linspace(Z_MIN, Z_MAX, GRID_SIZE)
    knots = jnp.linspace(Z_MIN, Z_MAX, NUM_BASIS)
    d = jnp.abs(z_grid[:, None] - knots[None, :])
    dx = (Z_MAX - Z_MIN) / (NUM_BASIS - 1)
    H = jnp.clip(1.0 - d / dx, 0.0, None)
    H = H / (H.sum(axis=1, keepdims=True) + 1e-08)
    dz = z_grid[1] - z_grid[0]
    I = jnp.cumsum(H * dz, axis=0)
    I_max = I[-1, :]
    I_max = jnp.where(I_max <= 0, 1.0, I_max)
    I = I / I_max[None, :]
    I_T = I.astype(jnp.float32).T
    return jnp.pad(I_T, ((0, 0), (0, TABLE_PAD - GRID_SIZE)))


def _table_body(ig_ref, rw_ref, pb_ref, out_ref):
    w = jax.nn.softplus(rw_ref[...])
    s = jnp.sum(ig_ref[...] * w, axis=0, keepdims=True)
    out_ref[...] = s * pb_ref[0] + pb_ref[1]


def _build_table(ig_T, raw_weights, beta, bias):
    pb = jnp.concatenate([beta.reshape(1), bias.reshape(1)]).astype(jnp.float32)
    out = pl.pallas_call(
        _table_body,
        out_shape=jax.ShapeDtypeStruct((1, TABLE_PAD), jnp.float32),
        in_specs=[
            pl.BlockSpec(memory_space=pltpu.VMEM),
            pl.BlockSpec(memory_space=pltpu.VMEM),
            pl.BlockSpec(memory_space=pltpu.SMEM),
        ],
        out_specs=pl.BlockSpec(memory_space=pltpu.VMEM),
    )(ig_T, raw_weights.reshape(NUM_BASIS, 1), pb)
    return out.reshape(TABLE_PAD)


def _make_sc_kernel(n_total):
    info = plsc.get_sparse_core_info()
    nc, ns, nl = info.num_cores, info.num_subcores, info.num_lanes
    nw = nc * ns
    per_w = n_total // nw
    chunk = 16384
    nchunk = per_w // chunk
    mesh = plsc.VectorSubcoreMesh(core_axis_name="c", subcore_axis_name="s")

    @functools.partial(
        pl.kernel,
        mesh=mesh,
        out_type=jax.ShapeDtypeStruct((n_total,), jnp.float32),
        scratch_types=[
            pltpu.VMEM((TABLE_PAD,), jnp.float32),
            pltpu.VMEM((nl,), jnp.float32),
            pltpu.VMEM((chunk,), jnp.float32),
            pltpu.VMEM((chunk,), jnp.float32),
            pltpu.VMEM((chunk,), jnp.float32),
            pltpu.VMEM((chunk,), jnp.float32),
            pltpu.SemaphoreType.DMA,
            pltpu.SemaphoreType.DMA,
            pltpu.SemaphoreType.DMA,
            pltpu.SemaphoreType.DMA,
        ],
        compiler_params=pltpu.CompilerParams(needs_layout_passes=False),
    )
    def sck(tab_hbm, alpha_hbm, z_hbm, out_hbm, tab_v, alpha_v,
            zb0, zb1, ob0, ob1, is0, is1, os0, os1):
        wid = lax.axis_index("s") * nc + lax.axis_index("c")
        base = wid * per_w
        pltpu.sync_copy(tab_hbm, tab_v)
        pltpu.sync_copy(alpha_hbm, alpha_v)
        av = alpha_v[...]
        zbufs, obufs = (zb0, zb1), (ob0, ob1)
        isems, osems = (is0, is1), (os0, os1)

        def compute(zb, ob):
            @plsc.parallel_loop(0, chunk // nl, unroll=8)
            def _body(i):
                zv = zb[pl.ds(i * nl, nl)]
                zc = jnp.minimum(jnp.maximum(zv, Z_MIN), Z_MAX)
                u = jnp.minimum((zc - Z_MIN) * U_SCALE, U_MAX)
                idx = u.astype(jnp.int32)
                fr = u - idx.astype(jnp.float32)
                t0 = plsc.load_gather(tab_v, [idx])
                t1 = plsc.load_gather(tab_v, [idx + 1])
                g = av * zc + t0 + fr * (t1 - t0)
                ob[pl.ds(i * nl, nl)] = 1.0 / (1.0 + jnp.exp(-g))

        in_h = [None, None]
        out_h = [None, None]
        in_h[0] = pltpu.async_copy(z_hbm.at[pl.ds(base, chunk)], zb0, is0)
        for c in range(nchunk):
            b = c & 1
            off = base + c * chunk
            in_h[b].wait()
            if c + 1 < nchunk:
                nb = (c + 1) & 1
                in_h[nb] = pltpu.async_copy(
                    z_hbm.at[pl.ds(base + (c + 1) * chunk, chunk)],
                    zbufs[nb], isems[nb])
            if c >= 2:
                out_h[b].wait()
            compute(zbufs[b], obufs[b])
            out_h[b] = pltpu.async_copy(
                obufs[b], out_hbm.at[pl.ds(off, chunk)], osems[b])
        out_h[(nchunk - 2) & 1].wait()
        out_h[(nchunk - 1) & 1].wait()

    return sck


def kernel(z, raw_weights, alpha, beta, bias):
    orig_shape = z.shape
    n_total = int(np.prod(orig_shape))
    table = _build_table(_make_I_grid_T(), raw_weights, beta, bias)
    alpha_vec = jnp.full((16,), alpha, dtype=jnp.float32)
    sck = _make_sc_kernel(n_total)
    out = sck(table, alpha_vec, z.reshape(n_total))
    return out.reshape(orig_shape)


# speedup vs baseline: 487.6521x; 1.3488x over previous
import functools

import jax
import jax.numpy as jnp
import numpy as np
from jax import lax
from jax.experimental import pallas as pl
from jax.experimental.pallas import tpu as pltpu
from jax.experimental.pallas import tpu_sc as plsc

NUM_BASIS = 16
GRID_SIZE = 1000
Z_MIN = -8.0
Z_MAX = 8.0
TABLE_PAD = 1024

U_SCALE = np.float32((GRID_SIZE - 1) / (Z_MAX - Z_MIN + 1e-8))
U_MAX = np.float32(GRID_SIZE - 1 - 1e-6)


def _make_I_grid_T():
    z_grid = jnp.linspace(Z_MIN, Z_MAX, GRID_SIZE)
    knots = jnp.linspace(Z_MIN, Z_MAX, NUM_BASIS)
    d = jnp.abs(z_grid[:, None] - knots[None, :])
    dx = (Z_MAX - Z_MIN) / (NUM_BASIS - 1)
    H = jnp.clip(1.0 - d / dx, 0.0, None)
    H = H / (H.sum(axis=1, keepdims=True) + 1e-08)
    dz = z_grid[1] - z_grid[0]
    I = jnp.cumsum(H * dz, axis=0)
    I_max = I[-1, :]
    I_max = jnp.where(I_max <= 0, 1.0, I_max)
    I = I / I_max[None, :]
    I_T = I.astype(jnp.float32).T
    return jnp.pad(I_T, ((0, 0), (0, TABLE_PAD - GRID_SIZE)))


def _table_body(ig_ref, rw_ref, pb_ref, out_ref):
    w = jax.nn.softplus(rw_ref[...])
    s = jnp.sum(ig_ref[...] * w, axis=0, keepdims=True)
    out_ref[...] = s * pb_ref[0] + pb_ref[1]


def _build_table(ig_T, raw_weights, beta, bias):
    pb = jnp.concatenate([beta.reshape(1), bias.reshape(1)]).astype(jnp.float32)
    out = pl.pallas_call(
        _table_body,
        out_shape=jax.ShapeDtypeStruct((1, TABLE_PAD), jnp.float32),
        in_specs=[
            pl.BlockSpec(memory_space=pltpu.VMEM),
            pl.BlockSpec(memory_space=pltpu.VMEM),
            pl.BlockSpec(memory_space=pltpu.SMEM),
        ],
        out_specs=pl.BlockSpec(memory_space=pltpu.VMEM),
    )(ig_T, raw_weights.reshape(NUM_BASIS, 1), pb)
    return out.reshape(TABLE_PAD)


def _make_sc_kernel(n_rows, n_cols):
    info = plsc.get_sparse_core_info()
    nc, ns, nl = info.num_cores, info.num_subcores, info.num_lanes
    nw = nc * ns
    rows_per_w = n_rows // nw
    crows = 16
    nchunk = rows_per_w // crows
    vecs_per_row = n_cols // nl
    mesh = plsc.VectorSubcoreMesh(core_axis_name="c", subcore_axis_name="s")

    @functools.partial(
        pl.kernel,
        mesh=mesh,
        out_type=jax.ShapeDtypeStruct((n_rows, n_cols), jnp.float32),
        scratch_types=[
            pltpu.VMEM((TABLE_PAD,), jnp.float32),
            pltpu.VMEM((nl,), jnp.float32),
            pltpu.VMEM((crows, n_cols), jnp.float32),
            pltpu.VMEM((crows, n_cols), jnp.float32),
            pltpu.VMEM((crows, n_cols), jnp.float32),
            pltpu.VMEM((crows, n_cols), jnp.float32),
            pltpu.SemaphoreType.DMA,
            pltpu.SemaphoreType.DMA,
            pltpu.SemaphoreType.DMA,
            pltpu.SemaphoreType.DMA,
        ],
        compiler_params=pltpu.CompilerParams(needs_layout_passes=False),
    )
    def sck(tab_hbm, alpha_hbm, z_hbm, out_hbm, tab_v, alpha_v,
            zb0, zb1, ob0, ob1, is0, is1, os0, os1):
        wid = lax.axis_index("s") * nc + lax.axis_index("c")
        base = wid * rows_per_w
        pltpu.sync_copy(tab_hbm, tab_v)
        pltpu.sync_copy(alpha_hbm, alpha_v)
        av = alpha_v[...]
        zbufs, obufs = (zb0, zb1), (ob0, ob1)
        isems, osems = (is0, is1), (os0, os1)

        def compute(zb, ob):
            @plsc.parallel_loop(0, crows * vecs_per_row, unroll=8)
            def _body(i):
                r = i // vecs_per_row
                col = (i % vecs_per_row) * nl
                zv = zb[r, pl.ds(col, nl)]
                zc = jnp.minimum(jnp.maximum(zv, Z_MIN), Z_MAX)
                u = jnp.minimum((zc - Z_MIN) * U_SCALE, U_MAX)
                idx = u.astype(jnp.int32)
                fr = u - idx.astype(jnp.float32)
                t0 = plsc.load_gather(tab_v, [idx])
                t1 = plsc.load_gather(tab_v, [idx + 1])
                g = av * zc + t0 + fr * (t1 - t0)
                ob[r, pl.ds(col, nl)] = 1.0 / (1.0 + jnp.exp(-g))

        in_h = [None, None]
        out_h = [None, None]
        in_h[0] = pltpu.async_copy(
            z_hbm.at[pl.ds(base, crows), :], zb0, is0)
        for c in range(nchunk):
            b = c & 1
            r0 = base + c * crows
            in_h[b].wait()
            if c + 1 < nchunk:
                nb = (c + 1) & 1
                in_h[nb] = pltpu.async_copy(
                    z_hbm.at[pl.ds(base + (c + 1) * crows, crows), :],
                    zbufs[nb], isems[nb])
            if c >= 2:
                out_h[b].wait()
            compute(zbufs[b], obufs[b])
            out_h[b] = pltpu.async_copy(
                obufs[b], out_hbm.at[pl.ds(r0, crows), :], osems[b])
        out_h[(nchunk - 2) & 1].wait()
        out_h[(nchunk - 1) & 1].wait()

    return sck


def kernel(z, raw_weights, alpha, beta, bias):
    orig_shape = z.shape
    n_total = int(np.prod(orig_shape))
    n_cols = orig_shape[-1]
    n_rows = n_total // n_cols
    table = _build_table(_make_I_grid_T(), raw_weights, beta, bias)
    alpha_vec = jnp.full((16,), alpha, dtype=jnp.float32)
    sck = _make_sc_kernel(n_rows, n_cols)
    out = sck(table, alpha_vec, z.reshape(n_rows, n_cols))
    return out.reshape(orig_shape)


# speedup vs baseline: 523.7495x; 1.0740x over previous
import functools

import jax
import jax.numpy as jnp
import numpy as np
from jax import lax
from jax.experimental import pallas as pl
from jax.experimental.pallas import tpu as pltpu
from jax.experimental.pallas import tpu_sc as plsc

NUM_BASIS = 16
GRID_SIZE = 1000
Z_MIN = -8.0
Z_MAX = 8.0
GRID_PAD = 1024
TABLE_PAD = 1040

U_SCALE = np.float32((GRID_SIZE - 1) / (Z_MAX - Z_MIN + 1e-8))


def _make_I_grid_T_flat():
    z_grid = jnp.linspace(Z_MIN, Z_MAX, GRID_SIZE)
    knots = jnp.linspace(Z_MIN, Z_MAX, NUM_BASIS)
    d = jnp.abs(z_grid[:, None] - knots[None, :])
    dx = (Z_MAX - Z_MIN) / (NUM_BASIS - 1)
    H = jnp.clip(1.0 - d / dx, 0.0, None)
    H = H / (H.sum(axis=1, keepdims=True) + 1e-08)
    dz = z_grid[1] - z_grid[0]
    I = jnp.cumsum(H * dz, axis=0)
    I_max = I[-1, :]
    I_max = jnp.where(I_max <= 0, 1.0, I_max)
    I = I / I_max[None, :]
    I_T = I.astype(jnp.float32).T
    return jnp.pad(I_T, ((0, 0), (0, GRID_PAD - GRID_SIZE))).reshape(-1)


def _make_sc_kernel(n_rows, n_cols):
    info = plsc.get_sparse_core_info()
    nc, ns, nl = info.num_cores, info.num_subcores, info.num_lanes
    nw = nc * ns
    rows_per_w = n_rows // nw
    crows = 16
    nchunk = rows_per_w // crows
    vecs_per_row = n_cols // nl
    nblk = GRID_PAD // nl
    mesh = plsc.VectorSubcoreMesh(core_axis_name="c", subcore_axis_name="s")

    @functools.partial(
        pl.kernel,
        mesh=mesh,
        out_type=jax.ShapeDtypeStruct((n_rows, n_cols), jnp.float32),
        scratch_types=[
            pltpu.VMEM((NUM_BASIS * GRID_PAD,), jnp.float32),
            pltpu.VMEM((nl,), jnp.float32),
            pltpu.VMEM((nl,), jnp.float32),
            pltpu.VMEM((TABLE_PAD,), jnp.float32),
            pltpu.VMEM((GRID_PAD,), jnp.float32),
            pltpu.VMEM((crows, n_cols), jnp.float32),
            pltpu.VMEM((crows, n_cols), jnp.float32),
            pltpu.VMEM((crows, n_cols), jnp.float32),
            pltpu.VMEM((crows, n_cols), jnp.float32),
            pltpu.SemaphoreType.DMA,
            pltpu.SemaphoreType.DMA,
            pltpu.SemaphoreType.DMA,
            pltpu.SemaphoreType.DMA,
        ],
        compiler_params=pltpu.CompilerParams(needs_layout_passes=False),
    )
    def sck(ig_hbm, params_hbm, rw_hbm, z_hbm, out_hbm,
            ig_v, par_v, w_v, t0_v, t1_v,
            zb0, zb1, ob0, ob1, is0, is1, os0, os1):
        wid = lax.axis_index("s") * nc + lax.axis_index("c")
        base = wid * rows_per_w
        zbufs, obufs = (zb0, zb1), (ob0, ob1)
        isems, osems = (is0, is1), (os0, os1)

        in_h = [None, None]
        out_h = [None, None]
        in_h[0] = pltpu.async_copy(
            z_hbm.at[pl.ds(base, crows), :], zb0, is0)

        pltpu.sync_copy(params_hbm, par_v)
        pltpu.sync_copy(rw_hbm, w_v)
        pltpu.sync_copy(ig_hbm, ig_v)

        x = w_v[...]
        q = jnp.exp(-jnp.abs(x))
        a1 = 1.0 + q
        L = (2.0 * q) / (2.0 + q)
        L = L + a1 * jnp.exp(-L) - 1.0
        L = L + a1 * jnp.exp(-L) - 1.0
        L = L + a1 * jnp.exp(-L) - 1.0
        w_v[...] = jnp.maximum(x, 0.0) + L

        def splat(ref, j):
            return plsc.load_gather(ref, [jnp.full((nl,), j, jnp.int32)])

        an = splat(par_v, 0)
        bn = splat(par_v, 1)
        cn = splat(par_v, 2)
        wspl = [splat(w_v, m) for m in range(NUM_BASIS)]

        @plsc.parallel_loop(0, nblk, unroll=2)
        def _tbl(b):
            acc = wspl[0] * ig_v[pl.ds(b * nl, nl)]
            for m in range(1, NUM_BASIS):
                acc = acc + wspl[m] * ig_v[pl.ds(m * GRID_PAD + b * nl, nl)]
            t0_v[pl.ds(b * nl, nl)] = cn + bn * acc

        t0_v[pl.ds(GRID_PAD, TABLE_PAD - GRID_PAD)] = jnp.zeros(
            (TABLE_PAD - GRID_PAD,), jnp.float32)

        @plsc.parallel_loop(0, nblk, unroll=2)
        def _shift(b):
            t1_v[pl.ds(b * nl, nl)] = t0_v[pl.ds(b * nl + 1, nl)]

        def compute(zb, ob):
            @plsc.parallel_loop(0, crows * vecs_per_row, unroll=8)
            def _body(i):
                r = i // vecs_per_row
                col = (i % vecs_per_row) * nl
                zv = zb[r, pl.ds(col, nl)]
                zc = jnp.minimum(jnp.maximum(zv, Z_MIN), Z_MAX)
                u = (zc - Z_MIN) * U_SCALE
                idx = u.astype(jnp.int32)
                fr = u - idx.astype(jnp.float32)
                t0 = plsc.load_gather(t0_v, [idx])
                t1 = plsc.load_gather(t1_v, [idx])
                gneg = an * zc + (t0 + fr * (t1 - t0))
                ob[r, pl.ds(col, nl)] = 1.0 / (1.0 + jnp.exp(gneg))

        for c in range(nchunk):
            b = c & 1
            r0 = base + c * crows
            in_h[b].wait()
            if c + 1 < nchunk:
                nb = (c + 1) & 1
                in_h[nb] = pltpu.async_copy(
                    z_hbm.at[pl.ds(base + (c + 1) * crows, crows), :],
                    zbufs[nb], isems[nb])
            if c >= 2:
                out_h[b].wait()
            compute(zbufs[b], obufs[b])
            out_h[b] = pltpu.async_copy(
                obufs[b], out_hbm.at[pl.ds(r0, crows), :], osems[b])
        out_h[(nchunk - 2) & 1].wait()
        out_h[(nchunk - 1) & 1].wait()

    return sck


def kernel(z, raw_weights, alpha, beta, bias):
    orig_shape = z.shape
    n_total = int(np.prod(orig_shape))
    n_cols = orig_shape[-1]
    n_rows = n_total // n_cols
    params = jnp.concatenate([
        (-alpha).reshape(1), (-beta).reshape(1), (-bias).reshape(1),
        jnp.zeros((13,), jnp.float32)]).astype(jnp.float32)
    sck = _make_sc_kernel(n_rows, n_cols)
    out = sck(_make_I_grid_T_flat(), params, raw_weights.astype(jnp.float32),
              z.reshape(n_rows, n_cols))
    return out.reshape(orig_shape)
